# baseline (device time: 55833 ns/iter reference)
import jax
import jax.numpy as jnp
from jax import lax
from jax.experimental import pallas as pl
from jax.experimental.pallas import tpu as pltpu

N_DEV = 8
M = 1536
N = 1536
GROUPS = 3
GC = N // GROUPS
CH = M // N_DEV

_MESH = pl.DeviceIdType.MESH


def kernel(A, B):
    def body(a_ref, b_ref, out_ref, acc_ref, recv_ref,
             rs_send, rs_recv, ag_send, ag_recv):
        my = lax.axis_index("i")
        r4 = lax.rem(my, 4)
        b1 = ((r4 >= 1) & (r4 <= 2)).astype(jnp.int32)
        b2 = (r4 >= 2).astype(jnp.int32)
        b3 = (my >= 4).astype(jnp.int32)
        p1 = my + 1 - 2 * lax.rem(my, 2)
        p2 = my + 3 - 2 * r4
        p3 = lax.rem(my + 4, N_DEV)
        partners = [p1, p2, p3]
        bits = [b1, b2, b3]

        barrier_sem = pltpu.get_barrier_semaphore()
        for p in partners:
            pl.semaphore_signal(barrier_sem, inc=1, device_id=(p,),
                                device_id_type=_MESH)
        pl.semaphore_wait(barrier_sem, 3)

        a_bf = a_ref[:, :].astype(jnp.bfloat16)

        starts = [jnp.int32(0)] * GROUPS
        rdmas = []
        half = M // 2
        for g in range(GROUPS):
            acc_ref[:, pl.ds(g * GC, GC)] = jnp.dot(
                a_bf, b_ref[:, pl.ds(g * GC, GC)].astype(jnp.bfloat16),
                preferred_element_type=jnp.float32,
            ).astype(jnp.bfloat16)
            p, b = partners[g], bits[g]
            keep = b * half
            send = (1 - b) * half
            rdma = pltpu.make_async_remote_copy(
                src_ref=acc_ref.at[pl.ds(send, half), pl.ds(g * GC, GC)],
                dst_ref=recv_ref.at[pl.ds(0, half), pl.ds(g * GC, GC)],
                send_sem=rs_send.at[0, g], recv_sem=rs_recv.at[0, g],
                device_id=(p,), device_id_type=_MESH)
            rdma.start()
            rdmas.append((rdma, keep))
            starts[g] = keep
        for s in range(3):
            if s > 0:
                half = (M // 2) >> s
                rdmas = []
                for g in range(GROUPS):
                    d = (g + s) % 3
                    p, b = partners[d], bits[d]
                    keep = starts[g] + b * half
                    send = starts[g] + (1 - b) * half
                    rdma = pltpu.make_async_remote_copy(
                        src_ref=acc_ref.at[pl.ds(send, half), pl.ds(g * GC, GC)],
                        dst_ref=recv_ref.at[pl.ds(0, half), pl.ds(g * GC, GC)],
                        send_sem=rs_send.at[s, g], recv_sem=rs_recv.at[s, g],
                        device_id=(p,), device_id_type=_MESH)
                    rdma.start()
                    rdmas.append((rdma, keep))
                    starts[g] = keep
            for g, (rdma, keep) in enumerate(rdmas):
                rdma.wait()
                acc_ref[pl.ds(keep, half), pl.ds(g * GC, GC)] += \
                    recv_ref[pl.ds(0, half), pl.ds(g * GC, GC)]

        for g in range(GROUPS):
            out_ref[pl.ds(starts[g], CH), pl.ds(g * GC, GC)] = jnp.maximum(
                acc_ref[pl.ds(starts[g], CH), pl.ds(g * GC, GC)], 0)

        for s in range(3):
            size = CH << s
            rdmas = []
            for g in range(GROUPS):
                d = (g + 2 - s) % 3
                p, b = partners[d], bits[d]
                rdma = pltpu.make_async_remote_copy(
                    src_ref=out_ref.at[pl.ds(starts[g], size), pl.ds(g * GC, GC)],
                    dst_ref=out_ref.at[pl.ds(starts[g], size), pl.ds(g * GC, GC)],
                    send_sem=ag_send.at[s, g], recv_sem=ag_recv.at[s, g],
                    device_id=(p,), device_id_type=_MESH)
                rdma.start()
                rdmas.append(rdma)
                starts[g] = starts[g] - b * size
            for rdma in rdmas:
                rdma.wait()

    return pl.pallas_call(
        body,
        out_shape=jax.ShapeDtypeStruct((M, N), jnp.bfloat16),
        in_specs=[
            pl.BlockSpec(memory_space=pltpu.VMEM),
            pl.BlockSpec(memory_space=pltpu.VMEM),
        ],
        out_specs=pl.BlockSpec(memory_space=pltpu.VMEM),
        scratch_shapes=[
            pltpu.VMEM((M, N), jnp.bfloat16),
            pltpu.VMEM((M // 2, N), jnp.bfloat16),
            pltpu.SemaphoreType.DMA((3, GROUPS)),
            pltpu.SemaphoreType.DMA((3, GROUPS)),
            pltpu.SemaphoreType.DMA((3, GROUPS)),
            pltpu.SemaphoreType.DMA((3, GROUPS)),
        ],
        compiler_params=pltpu.CompilerParams(collective_id=0),
    )(A, B)


# device time: 48517 ns/iter; 1.1508x vs baseline; 1.1508x over previous
import jax
import jax.numpy as jnp
from jax import lax
from jax.experimental import pallas as pl
from jax.experimental.pallas import tpu as pltpu

N_DEV = 8
M = 1536
N = 1536
GROUPS = 3
LANES = 2
GC = N // GROUPS
SC = GC // LANES
CH = M // N_DEV
HALF = (768, 384, 192)
SIZE = (192, 384, 768)
RS_OFF = (0, 768, 1152)

_MESH = pl.DeviceIdType.MESH


def kernel(A, B):
    def body(a_ref, b_ref, out_ref, acc_ref, recv_ref,
             rs_send, rs_recv, ag_send, ag_recv):
        my = lax.axis_index("i")
        r4 = lax.rem(my, 4)
        bits = [
            ((r4 >= 1) & (r4 <= 2)).astype(jnp.int32),
            (r4 >= 2).astype(jnp.int32),
            (my >= 4).astype(jnp.int32),
        ]
        partners = [
            my + 1 - 2 * lax.rem(my, 2),
            my + 3 - 2 * r4,
            lax.rem(my + 4, N_DEV),
        ]

        barrier_sem = pltpu.get_barrier_semaphore()
        for p in partners:
            pl.semaphore_signal(barrier_sem, inc=1, device_id=(p,),
                                device_id_type=_MESH)
        pl.semaphore_wait(barrier_sem, 3)

        rs_keep = [[None] * 3 for _ in range(GROUPS)]
        rs_send_start = [[None] * 3 for _ in range(GROUPS)]
        own_start = [None] * GROUPS
        ag_start = [[None] * 3 for _ in range(GROUPS)]
        for g in range(GROUPS):
            st = jnp.int32(0)
            for s in range(3):
                b = bits[(g + s) % 3]
                rs_keep[g][s] = st + b * HALF[s]
                rs_send_start[g][s] = st + (1 - b) * HALF[s]
                st = rs_keep[g][s]
            own_start[g] = st
            for s in range(3):
                ag_start[g][s] = st
                st = st - bits[(g + 2 - s) % 3] * SIZE[s]

        a_bf = a_ref[:, :].astype(jnp.bfloat16)

        def col(g, c):
            return pl.ds(g * GC + c * SC, SC)

        def rs_rdma(g, c, s):
            d = (g + s) % 3
            return pltpu.make_async_remote_copy(
                src_ref=acc_ref.at[pl.ds(rs_send_start[g][s], HALF[s]), col(g, c)],
                dst_ref=recv_ref.at[pl.ds(RS_OFF[s], HALF[s]), col(g, c)],
                send_sem=rs_send.at[s, g, c], recv_sem=rs_recv.at[s, g, c],
                device_id=(partners[d],), device_id_type=_MESH)

        def rs_accum(g, c, s):
            sl = (pl.ds(rs_keep[g][s], HALF[s]), col(g, c))
            acc_ref[sl] += recv_ref[pl.ds(RS_OFF[s], HALF[s]), col(g, c)]

        def ag_rdma(g, c, s):
            d = (g + 2 - s) % 3
            sl = (pl.ds(ag_start[g][s], SIZE[s]), col(g, c))
            return pltpu.make_async_remote_copy(
                src_ref=out_ref.at[sl], dst_ref=out_ref.at[sl],
                send_sem=ag_send.at[s, g, c], recv_sem=ag_recv.at[s, g, c],
                device_id=(partners[d],), device_id_type=_MESH)

        rs_rd = {}
        for g in range(GROUPS):
            acc_ref[:, pl.ds(g * GC, GC)] = jnp.dot(
                a_bf, b_ref[:, pl.ds(g * GC, GC)].astype(jnp.bfloat16),
                preferred_element_type=jnp.float32,
            ).astype(jnp.bfloat16)
            for c in range(LANES):
                rd = rs_rdma(g, c, 0)
                rd.start()
                rs_rd[(g, c, 0)] = rd

        for s in (1, 2):
            for c in range(LANES):
                for g in range(GROUPS):
                    rs_rd[(g, c, s - 1)].wait()
                    rs_accum(g, c, s - 1)
                    rd = rs_rdma(g, c, s)
                    rd.start()
                    rs_rd[(g, c, s)] = rd

        ag_rd = {}
        for c in range(LANES):
            for g in range(GROUPS):
                rs_rd[(g, c, 2)].wait()
                rs_accum(g, c, 2)
                sl = (pl.ds(own_start[g], CH), col(g, c))
                out_ref[sl] = jnp.maximum(acc_ref[sl], 0)
                rd = ag_rdma(g, c, 0)
                rd.start()
                ag_rd[(g, c, 0)] = rd

        for s in (1, 2):
            for c in range(LANES):
                for g in range(GROUPS):
                    ag_rd[(g, c, s - 1)].wait()
                    rd = ag_rdma(g, c, s)
                    rd.start()
                    ag_rd[(g, c, s)] = rd
        for c in range(LANES):
            for g in range(GROUPS):
                ag_rd[(g, c, 2)].wait()

    return pl.pallas_call(
        body,
        out_shape=jax.ShapeDtypeStruct((M, N), jnp.bfloat16),
        in_specs=[
            pl.BlockSpec(memory_space=pltpu.VMEM),
            pl.BlockSpec(memory_space=pltpu.VMEM),
        ],
        out_specs=pl.BlockSpec(memory_space=pltpu.VMEM),
        scratch_shapes=[
            pltpu.VMEM((M, N), jnp.bfloat16),
            pltpu.VMEM((1344, N), jnp.bfloat16),
            pltpu.SemaphoreType.DMA((3, GROUPS, LANES)),
            pltpu.SemaphoreType.DMA((3, GROUPS, LANES)),
            pltpu.SemaphoreType.DMA((3, GROUPS, LANES)),
            pltpu.SemaphoreType.DMA((3, GROUPS, LANES)),
        ],
        compiler_params=pltpu.CompilerParams(collective_id=0),
    )(A, B)


# device time: 46366 ns/iter; 1.2042x vs baseline; 1.0464x over previous
import jax
import jax.numpy as jnp
from jax import lax
from jax.experimental import pallas as pl
from jax.experimental.pallas import tpu as pltpu

N_DEV = 8
M = 1536
N = 1536
GROUPS = 3
LANES = 4
GC = N // GROUPS
SC = GC // LANES
CH = M // N_DEV
HALF = (768, 384, 192)
SIZE = (192, 384, 768)
RS_OFF = (0, 768, 1152)

_MESH = pl.DeviceIdType.MESH


def kernel(A, B):
    def body(a_ref, b_ref, out_ref, acc_ref, recv_ref,
             rs_send, rs_recv, ag_send, ag_recv):
        my = lax.axis_index("i")
        r4 = lax.rem(my, 4)
        bits = [
            ((r4 >= 1) & (r4 <= 2)).astype(jnp.int32),
            (r4 >= 2).astype(jnp.int32),
            (my >= 4).astype(jnp.int32),
        ]
        partners = [
            my + 1 - 2 * lax.rem(my, 2),
            my + 3 - 2 * r4,
            lax.rem(my + 4, N_DEV),
        ]

        barrier_sem = pltpu.get_barrier_semaphore()
        for p in partners:
            pl.semaphore_signal(barrier_sem, inc=1, device_id=(p,),
                                device_id_type=_MESH)
        pl.semaphore_wait(barrier_sem, 3)

        rs_keep = [[None] * 3 for _ in range(GROUPS)]
        rs_send_start = [[None] * 3 for _ in range(GROUPS)]
        own_start = [None] * GROUPS
        ag_start = [[None] * 3 for _ in range(GROUPS)]
        for g in range(GROUPS):
            st = jnp.int32(0)
            for s in range(3):
                b = bits[(g + s) % 3]
                rs_keep[g][s] = st + b * HALF[s]
                rs_send_start[g][s] = st + (1 - b) * HALF[s]
                st = rs_keep[g][s]
            own_start[g] = st
            for s in range(3):
                ag_start[g][s] = st
                st = st - bits[(g + 2 - s) % 3] * SIZE[s]

        b_bf = b_ref[:, :].astype(jnp.bfloat16)

        def col(g, c):
            return pl.ds(g * GC + c * SC, SC)

        def rs_rdma(g, c, s):
            d = (g + s) % 3
            return pltpu.make_async_remote_copy(
                src_ref=acc_ref.at[pl.ds(rs_send_start[g][s], HALF[s]), col(g, c)],
                dst_ref=recv_ref.at[pl.ds(RS_OFF[s], HALF[s]), col(g, c)],
                send_sem=rs_send.at[s, g, c], recv_sem=rs_recv.at[s, g, c],
                device_id=(partners[d],), device_id_type=_MESH)

        def rs_accum(g, c, s):
            sl = (pl.ds(rs_keep[g][s], HALF[s]), col(g, c))
            acc_ref[sl] += recv_ref[pl.ds(RS_OFF[s], HALF[s]), col(g, c)]

        def ag_rdma(g, c, s):
            d = (g + 2 - s) % 3
            sl = (pl.ds(ag_start[g][s], SIZE[s]), col(g, c))
            return pltpu.make_async_remote_copy(
                src_ref=out_ref.at[sl], dst_ref=out_ref.at[sl],
                send_sem=ag_send.at[s, g, c], recv_sem=ag_recv.at[s, g, c],
                device_id=(partners[d],), device_id_type=_MESH)

        def partial(rows, g):
            return jnp.dot(
                a_ref[rows, :].astype(jnp.bfloat16),
                b_bf[:, g * GC:(g + 1) * GC],
                preferred_element_type=jnp.float32,
            ).astype(jnp.bfloat16)

        rs_rd = {}
        for g in range(GROUPS):
            rows = pl.ds(rs_send_start[g][0], HALF[0])
            acc_ref[rows, pl.ds(g * GC, GC)] = partial(rows, g)
            for c in range(LANES):
                rd = rs_rdma(g, c, 0)
                rd.start()
                rs_rd[(g, c, 0)] = rd
        for g in range(GROUPS):
            rows = pl.ds(rs_keep[g][0], HALF[0])
            acc_ref[rows, pl.ds(g * GC, GC)] = partial(rows, g)

        for s in (1, 2):
            for c in range(LANES):
                for g in range(GROUPS):
                    rs_rd[(g, c, s - 1)].wait()
                    rs_accum(g, c, s - 1)
                    rd = rs_rdma(g, c, s)
                    rd.start()
                    rs_rd[(g, c, s)] = rd

        ag_rd = {}
        for c in range(LANES):
            for g in range(GROUPS):
                rs_rd[(g, c, 2)].wait()
                rs_accum(g, c, 2)
                sl = (pl.ds(own_start[g], CH), col(g, c))
                out_ref[sl] = jnp.maximum(acc_ref[sl], 0)
                rd = ag_rdma(g, c, 0)
                rd.start()
                ag_rd[(g, c, 0)] = rd

        for s in (1, 2):
            for c in range(LANES):
                for g in range(GROUPS):
                    ag_rd[(g, c, s - 1)].wait()
                    rd = ag_rdma(g, c, s)
                    rd.start()
                    ag_rd[(g, c, s)] = rd
        for c in range(LANES):
            for g in range(GROUPS):
                ag_rd[(g, c, 2)].wait()

    return pl.pallas_call(
        body,
        out_shape=jax.ShapeDtypeStruct((M, N), jnp.bfloat16),
        in_specs=[
            pl.BlockSpec(memory_space=pltpu.VMEM),
            pl.BlockSpec(memory_space=pltpu.VMEM),
        ],
        out_specs=pl.BlockSpec(memory_space=pltpu.VMEM),
        scratch_shapes=[
            pltpu.VMEM((M, N), jnp.bfloat16),
            pltpu.VMEM((1344, N), jnp.bfloat16),
            pltpu.SemaphoreType.DMA((3, GROUPS, LANES)),
            pltpu.SemaphoreType.DMA((3, GROUPS, LANES)),
            pltpu.SemaphoreType.DMA((3, GROUPS, LANES)),
            pltpu.SemaphoreType.DMA((3, GROUPS, LANES)),
        ],
        compiler_params=pltpu.CompilerParams(collective_id=0),
    )(A, B)
